# baseline (speedup 1.0000x reference)
import functools

import jax
import jax.numpy as jnp
from jax import lax
from jax.experimental import pallas as pl
from jax.experimental.pallas import tpu as pltpu
from jax.experimental.pallas import tpu_sc as plsc

N_NODES = 10000
N_EDGES = 320000
D_FEAT = 128
D_EDGE = 16
H = 256
D_OUT = 128

NC = 2
NS = 16
NW = NC * NS
DP = D_FEAT // 2

K_CH = 5
C_E = N_EDGES // K_CH
PT = C_E // NW
GCH = 80
G_STEPS = PT // GCH
SCH = 80
S_STEPS = PT // SCH

EBLK = 2000
NBLK = 1000

ROWS_MAIN = 640
ROWS_TAIL = N_NODES - 15 * ROWS_MAIN

_mesh = plsc.VectorSubcoreMesh(core_axis_name="c", subcore_axis_name="s")


def _make_gather(k):
  @functools.partial(
      pl.kernel,
      out_type=(
          jax.ShapeDtypeStruct((C_E, DP), jnp.int32),
          jax.ShapeDtypeStruct((C_E, DP), jnp.int32),
      ),
      mesh=_mesh,
      scratch_types=[
          pltpu.VMEM((PT,), jnp.int32),
          pltpu.VMEM((PT,), jnp.int32),
          pltpu.VMEM((GCH, DP), jnp.int32),
          pltpu.VMEM((GCH, DP), jnp.int32),
          pltpu.VMEM((GCH, DP), jnp.int32),
          pltpu.VMEM((GCH, DP), jnp.int32),
          pltpu.VMEM_SHARED((N_NODES, DP), jnp.int32),
          pltpu.SemaphoreType.DMA,
          pltpu.SemaphoreType.DMA,
          pltpu.SemaphoreType.DMA,
          pltpu.SemaphoreType.DMA,
      ],
      name=f"sc_gather_{k}",
      compiler_params=pltpu.CompilerParams(use_tc_tiling_on_sc=False),
  )
  def _sc_gather(nodes_hbm, recv_hbm, send_hbm, orecv_hbm, osend_hbm,
                 ridx_v, sidx_v, rb0, sb0, rb1, sb1, table,
                 mr0, ms0, mr1, ms1):
    cid = lax.axis_index("c")
    sid = lax.axis_index("s")
    wid = sid * NC + cid
    row0 = sid * ROWS_MAIN

    pltpu.sync_copy(recv_hbm.at[pl.ds(k * C_E + wid * PT, PT)], ridx_v)
    pltpu.sync_copy(send_hbm.at[pl.ds(k * C_E + wid * PT, PT)], sidx_v)

    @pl.when(sid < 15)
    def _():
        pltpu.sync_copy(nodes_hbm.at[pl.ds(row0, ROWS_MAIN)],
                        table.at[pl.ds(row0, ROWS_MAIN)])

    @pl.when(sid == 15)
    def _():
        pltpu.sync_copy(nodes_hbm.at[pl.ds(15 * ROWS_MAIN, ROWS_TAIL)],
                        table.at[pl.ds(15 * ROWS_MAIN, ROWS_TAIL)])

    plsc.subcore_barrier()

    def gath(j, rb, sb):
        pltpu.sync_copy(table.at[ridx_v.at[pl.ds(j * GCH, GCH)]], rb)
        pltpu.sync_copy(table.at[sidx_v.at[pl.ds(j * GCH, GCH)]], sb)

    def wr_start(j, rb, sb, mr, ms):
        base = wid * PT + j * GCH
        cr = pltpu.async_copy(rb, orecv_hbm.at[pl.ds(base, GCH)], mr)
        cs = pltpu.async_copy(sb, osend_hbm.at[pl.ds(base, GCH)], ms)
        return cr, cs

    gath(0, rb0, sb0)

    @pl.loop(0, (G_STEPS - 1) // 2)
    def _(t):
        j = 2 * t
        c0 = wr_start(j, rb0, sb0, mr0, ms0)
        gath(j + 1, rb1, sb1)
        c0[0].wait()
        c0[1].wait()
        c1 = wr_start(j + 1, rb1, sb1, mr1, ms1)
        gath(j + 2, rb0, sb0)
        c1[0].wait()
        c1[1].wait()

    base = wid * PT + (G_STEPS - 1) * GCH
    pltpu.sync_copy(rb0, orecv_hbm.at[pl.ds(base, GCH)])
    pltpu.sync_copy(sb0, osend_hbm.at[pl.ds(base, GCH)])

  return _sc_gather


_GATHERS = [_make_gather(k) for k in range(K_CH)]


def _make_scatter(k):
  @functools.partial(
      pl.kernel,
      out_type=jax.ShapeDtypeStruct((NC, N_NODES, D_OUT), jnp.float32),
      mesh=_mesh,
      scratch_types=[
          pltpu.VMEM((SCH, D_OUT), jnp.float32),
          pltpu.VMEM((SCH, D_OUT), jnp.float32),
          pltpu.VMEM((S_STEPS, SCH), jnp.int32),
          pltpu.VMEM_SHARED((N_NODES, D_OUT), jnp.float32),
          pltpu.SemaphoreType.DMA,
          pltpu.SemaphoreType.DMA,
      ],
      name=f"sc_scatter_{k}",
  )
  def _sc_scatter(eo_hbm, ridx_hbm, zeros_hbm, out_hbm, b0, b1, idx2, shared,
                  m0, m1):
    cid = lax.axis_index("c")
    sid = lax.axis_index("s")
    wid = sid * NC + cid
    row0 = sid * ROWS_MAIN

    pltpu.sync_copy(ridx_hbm.at[k * NW + wid], idx2)

    @pl.when(sid < 15)
    def _():
        pltpu.sync_copy(zeros_hbm.at[pl.ds(row0, ROWS_MAIN)],
                        shared.at[pl.ds(row0, ROWS_MAIN)])

    @pl.when(sid == 15)
    def _():
        pltpu.sync_copy(zeros_hbm.at[pl.ds(15 * ROWS_MAIN, ROWS_TAIL)],
                        shared.at[pl.ds(15 * ROWS_MAIN, ROWS_TAIL)])

    plsc.subcore_barrier()

    def ld_start(j, b, m):
        return pltpu.async_copy(eo_hbm.at[pl.ds(wid * PT + j * SCH, SCH)],
                                b, m)

    ld_start(0, b0, m0).wait()

    @pl.loop(0, (S_STEPS - 1) // 2)
    def _(t):
        j = 2 * t
        c1 = ld_start(j + 1, b1, m1)
        pltpu.sync_copy(b0, shared.at[idx2.at[j]], add=True)
        c1.wait()
        c0 = ld_start(j + 2, b0, m0)
        pltpu.sync_copy(b1, shared.at[idx2.at[j + 1]], add=True)
        c0.wait()

    pltpu.sync_copy(b0, shared.at[idx2.at[S_STEPS - 1]], add=True)

    plsc.subcore_barrier()

    @pl.when(sid < 15)
    def _():
        pltpu.sync_copy(shared.at[pl.ds(row0, ROWS_MAIN)],
                        out_hbm.at[cid, pl.ds(row0, ROWS_MAIN)])

    @pl.when(sid == 15)
    def _():
        pltpu.sync_copy(shared.at[pl.ds(15 * ROWS_MAIN, ROWS_TAIL)],
                        out_hbm.at[cid, pl.ds(15 * ROWS_MAIN, ROWS_TAIL)])

  return _sc_scatter


_SCATTERS = [_make_scatter(k) for k in range(K_CH)]


def _unpack_pairs(w):
    lo = pltpu.bitcast(jnp.left_shift(w, 16), jnp.float32)
    hi = pltpu.bitcast(jnp.bitwise_and(w, jnp.int32(-65536)), jnp.float32)
    return lo.astype(jnp.bfloat16), hi.astype(jnp.bfloat16)


def _edge_mlp_body(ea, rv, sv, wa, wre, wro, wse, wso, b1, w2, b2, out):
    rlo, rhi = _unpack_pairs(rv[...])
    slo, shi = _unpack_pairs(sv[...])
    acc = jnp.dot(ea[...], wa[...], preferred_element_type=jnp.float32)
    acc += jnp.dot(rlo, wre[...], preferred_element_type=jnp.float32)
    acc += jnp.dot(rhi, wro[...], preferred_element_type=jnp.float32)
    acc += jnp.dot(slo, wse[...], preferred_element_type=jnp.float32)
    acc += jnp.dot(shi, wso[...], preferred_element_type=jnp.float32)
    h = jnp.maximum(acc + b1[...], 0.0).astype(jnp.bfloat16)
    out[...] = jnp.dot(h, w2[...], preferred_element_type=jnp.float32) + b2[...]


def _edge_mlp(k, ea_bf, recv, send, wa, wre, wro, wse, wso, b1, w2, b2):
    n_blocks = C_E // EBLK
    base = k * n_blocks
    return pl.pallas_call(
        _edge_mlp_body,
        grid=(n_blocks,),
        in_specs=[
            pl.BlockSpec((EBLK, D_EDGE), lambda i, base=base: (base + i, 0)),
            pl.BlockSpec((EBLK, DP), lambda i: (i, 0)),
            pl.BlockSpec((EBLK, DP), lambda i: (i, 0)),
            pl.BlockSpec((D_EDGE, H), lambda i: (0, 0)),
            pl.BlockSpec((DP, H), lambda i: (0, 0)),
            pl.BlockSpec((DP, H), lambda i: (0, 0)),
            pl.BlockSpec((DP, H), lambda i: (0, 0)),
            pl.BlockSpec((DP, H), lambda i: (0, 0)),
            pl.BlockSpec((1, H), lambda i: (0, 0)),
            pl.BlockSpec((H, D_OUT), lambda i: (0, 0)),
            pl.BlockSpec((1, D_OUT), lambda i: (0, 0)),
        ],
        out_specs=pl.BlockSpec((EBLK, D_OUT), lambda i: (i, 0)),
        out_shape=jax.ShapeDtypeStruct((C_E, D_OUT), jnp.float32),
    )(ea_bf, recv, send, wa, wre, wro, wse, wso, b1, w2, b2)


def _node_mlp_body(a0, a1, a2, a3, a4, nd, wna, wnb, b1, w2, b2, out):
    agg = ((a0[0] + a0[1]) + (a1[0] + a1[1]) + (a2[0] + a2[1])
           + (a3[0] + a3[1]) + (a4[0] + a4[1])).astype(jnp.bfloat16)
    acc = jnp.dot(agg, wna[...], preferred_element_type=jnp.float32)
    acc += jnp.dot(nd[...].astype(jnp.bfloat16), wnb[...],
                   preferred_element_type=jnp.float32)
    h = jnp.maximum(acc + b1[...], 0.0).astype(jnp.bfloat16)
    out[...] = jnp.dot(h, w2[...], preferred_element_type=jnp.float32) + b2[...]


def _node_mlp(aggs, nodes, wna, wnb, b1, w2, b2):
    n_blocks = N_NODES // NBLK
    agg_spec = pl.BlockSpec((NC, NBLK, D_OUT), lambda i: (0, i, 0))
    return pl.pallas_call(
        _node_mlp_body,
        grid=(n_blocks,),
        in_specs=[agg_spec] * K_CH + [
            pl.BlockSpec((NBLK, D_FEAT), lambda i: (i, 0)),
            pl.BlockSpec((D_OUT, H), lambda i: (0, 0)),
            pl.BlockSpec((D_FEAT, H), lambda i: (0, 0)),
            pl.BlockSpec((1, H), lambda i: (0, 0)),
            pl.BlockSpec((H, D_OUT), lambda i: (0, 0)),
            pl.BlockSpec((1, D_OUT), lambda i: (0, 0)),
        ],
        out_specs=pl.BlockSpec((NBLK, D_OUT), lambda i: (i, 0)),
        out_shape=jax.ShapeDtypeStruct((N_NODES, D_OUT), jnp.float32),
    )(*aggs, nodes, wna, wnb, b1, w2, b2)


def kernel(nodes, edge_attr, senders, receivers,
           W_e1, b_e1, W_e2, b_e2, W_n1, b_n1, W_n2, b_n2):
    recv_i = receivers.astype(jnp.int32)
    send_i = senders.astype(jnp.int32)
    ea_bf = edge_attr.astype(jnp.bfloat16)

    nodes_packed = jax.lax.bitcast_convert_type(
        nodes.astype(jnp.bfloat16).reshape(N_NODES, DP, 2), jnp.int32)
    wa = W_e1[:D_EDGE].astype(jnp.bfloat16)
    wr = W_e1[D_EDGE:D_EDGE + D_FEAT]
    ws = W_e1[D_EDGE + D_FEAT:]
    wre = wr[0::2].astype(jnp.bfloat16)
    wro = wr[1::2].astype(jnp.bfloat16)
    wse = ws[0::2].astype(jnp.bfloat16)
    wso = ws[1::2].astype(jnp.bfloat16)
    b1e = b_e1.reshape(1, H)
    w2e = W_e2.astype(jnp.bfloat16)
    b2e = b_e2.reshape(1, D_OUT)
    zeros = jnp.zeros((N_NODES, D_OUT), jnp.float32)

    ridx4 = recv_i.reshape(K_CH * NW, S_STEPS, SCH)
    eo_chunks = []
    agg_chunks = []
    for k in range(K_CH):
        recv_rows, send_rows = _GATHERS[k](nodes_packed, recv_i, send_i)
        eo = _edge_mlp(k, ea_bf, recv_rows, send_rows,
                       wa, wre, wro, wse, wso, b1e, w2e, b2e)
        eo_chunks.append(eo)
        agg_chunks.append(_SCATTERS[k](eo, ridx4, zeros))

    edge_out = jnp.concatenate(eo_chunks, axis=0)

    wna = W_n1[:D_OUT].astype(jnp.bfloat16)
    wnb = W_n1[D_OUT:].astype(jnp.bfloat16)
    node_out = _node_mlp(
        agg_chunks, nodes, wna, wnb, b_n1.reshape(1, H),
        W_n2.astype(jnp.bfloat16), b_n2.reshape(1, D_OUT))

    return (node_out, edge_out)


# speedup vs baseline: 1.6069x; 1.6069x over previous
import functools

import jax
import jax.numpy as jnp
from jax import lax
from jax.experimental import pallas as pl
from jax.experimental.pallas import tpu as pltpu
from jax.experimental.pallas import tpu_sc as plsc

N_NODES = 10000
N_EDGES = 320000
D_FEAT = 128
D_EDGE = 16
H = 256
D_OUT = 128

NC = 2
NS = 16
NW = NC * NS

K_CH = 5
C_E = N_EDGES // K_CH
PT = C_E // NW
GCH = 80
G_STEPS = PT // GCH
SCH = 80
S_STEPS = PT // SCH

EBLK = 2000
NBLK = 1000

ROWS_MAIN = 640
ROWS_TAIL = N_NODES - 15 * ROWS_MAIN

_mesh = plsc.VectorSubcoreMesh(core_axis_name="c", subcore_axis_name="s")


def _make_gather(k):
  @functools.partial(
      pl.kernel,
      out_type=(
          jax.ShapeDtypeStruct((C_E, D_FEAT), jnp.float32),
          jax.ShapeDtypeStruct((C_E, D_FEAT), jnp.float32),
      ),
      mesh=_mesh,
      scratch_types=[
          pltpu.VMEM((PT,), jnp.int32),
          pltpu.VMEM((PT,), jnp.int32),
          pltpu.VMEM((GCH, D_FEAT), jnp.float32),
          pltpu.VMEM((GCH, D_FEAT), jnp.float32),
          pltpu.VMEM((GCH, D_FEAT), jnp.float32),
          pltpu.VMEM((GCH, D_FEAT), jnp.float32),
          pltpu.VMEM_SHARED((N_NODES, D_FEAT), jnp.float32),
          pltpu.SemaphoreType.DMA,
          pltpu.SemaphoreType.DMA,
          pltpu.SemaphoreType.DMA,
          pltpu.SemaphoreType.DMA,
      ],
      name=f"sc_gather_{k}",
  )
  def _sc_gather(nodes_hbm, recv_hbm, send_hbm, orecv_hbm, osend_hbm,
                 ridx_v, sidx_v, rb0, sb0, rb1, sb1, table,
                 mr0, ms0, mr1, ms1):
    cid = lax.axis_index("c")
    sid = lax.axis_index("s")
    wid = sid * NC + cid
    row0 = sid * ROWS_MAIN

    pltpu.sync_copy(recv_hbm.at[pl.ds(k * C_E + wid * PT, PT)], ridx_v)
    pltpu.sync_copy(send_hbm.at[pl.ds(k * C_E + wid * PT, PT)], sidx_v)

    @pl.when(sid < 15)
    def _():
        pltpu.sync_copy(nodes_hbm.at[pl.ds(row0, ROWS_MAIN)],
                        table.at[pl.ds(row0, ROWS_MAIN)])

    @pl.when(sid == 15)
    def _():
        pltpu.sync_copy(nodes_hbm.at[pl.ds(15 * ROWS_MAIN, ROWS_TAIL)],
                        table.at[pl.ds(15 * ROWS_MAIN, ROWS_TAIL)])

    plsc.subcore_barrier()

    def gath(j, rb, sb):
        pltpu.sync_copy(table.at[ridx_v.at[pl.ds(j * GCH, GCH)]], rb)
        pltpu.sync_copy(table.at[sidx_v.at[pl.ds(j * GCH, GCH)]], sb)

    def wr_start(j, rb, sb, mr, ms):
        base = wid * PT + j * GCH
        cr = pltpu.async_copy(rb, orecv_hbm.at[pl.ds(base, GCH)], mr)
        cs = pltpu.async_copy(sb, osend_hbm.at[pl.ds(base, GCH)], ms)
        return cr, cs

    gath(0, rb0, sb0)

    @pl.loop(0, (G_STEPS - 1) // 2)
    def _(t):
        j = 2 * t
        c0 = wr_start(j, rb0, sb0, mr0, ms0)
        gath(j + 1, rb1, sb1)
        c0[0].wait()
        c0[1].wait()
        c1 = wr_start(j + 1, rb1, sb1, mr1, ms1)
        gath(j + 2, rb0, sb0)
        c1[0].wait()
        c1[1].wait()

    base = wid * PT + (G_STEPS - 1) * GCH
    pltpu.sync_copy(rb0, orecv_hbm.at[pl.ds(base, GCH)])
    pltpu.sync_copy(sb0, osend_hbm.at[pl.ds(base, GCH)])

  return _sc_gather


_GATHERS = [_make_gather(k) for k in range(K_CH)]


def _make_scatter(k):
  @functools.partial(
      pl.kernel,
      out_type=jax.ShapeDtypeStruct((NC, N_NODES, D_OUT), jnp.float32),
      mesh=_mesh,
      scratch_types=[
          pltpu.VMEM((SCH, D_OUT), jnp.float32),
          pltpu.VMEM((SCH, D_OUT), jnp.float32),
          pltpu.VMEM((S_STEPS, SCH), jnp.int32),
          pltpu.VMEM_SHARED((N_NODES, D_OUT), jnp.float32),
          pltpu.SemaphoreType.DMA,
          pltpu.SemaphoreType.DMA,
      ],
      name=f"sc_scatter_{k}",
  )
  def _sc_scatter(eo_hbm, ridx_hbm, zeros_hbm, out_hbm, b0, b1, idx2, shared,
                  m0, m1):
    cid = lax.axis_index("c")
    sid = lax.axis_index("s")
    wid = sid * NC + cid
    row0 = sid * ROWS_MAIN

    pltpu.sync_copy(ridx_hbm.at[k * NW + wid], idx2)

    @pl.when(sid < 15)
    def _():
        pltpu.sync_copy(zeros_hbm.at[pl.ds(row0, ROWS_MAIN)],
                        shared.at[pl.ds(row0, ROWS_MAIN)])

    @pl.when(sid == 15)
    def _():
        pltpu.sync_copy(zeros_hbm.at[pl.ds(15 * ROWS_MAIN, ROWS_TAIL)],
                        shared.at[pl.ds(15 * ROWS_MAIN, ROWS_TAIL)])

    plsc.subcore_barrier()

    def ld_start(j, b, m):
        return pltpu.async_copy(eo_hbm.at[pl.ds(wid * PT + j * SCH, SCH)],
                                b, m)

    ld_start(0, b0, m0).wait()

    @pl.loop(0, (S_STEPS - 1) // 2)
    def _(t):
        j = 2 * t
        c1 = ld_start(j + 1, b1, m1)
        pltpu.sync_copy(b0, shared.at[idx2.at[j]], add=True)
        c1.wait()
        c0 = ld_start(j + 2, b0, m0)
        pltpu.sync_copy(b1, shared.at[idx2.at[j + 1]], add=True)
        c0.wait()

    pltpu.sync_copy(b0, shared.at[idx2.at[S_STEPS - 1]], add=True)

    plsc.subcore_barrier()

    @pl.when(sid < 15)
    def _():
        pltpu.sync_copy(shared.at[pl.ds(row0, ROWS_MAIN)],
                        out_hbm.at[cid, pl.ds(row0, ROWS_MAIN)])

    @pl.when(sid == 15)
    def _():
        pltpu.sync_copy(shared.at[pl.ds(15 * ROWS_MAIN, ROWS_TAIL)],
                        out_hbm.at[cid, pl.ds(15 * ROWS_MAIN, ROWS_TAIL)])

  return _sc_scatter


_SCATTERS = [_make_scatter(k) for k in range(K_CH)]


def _edge_mlp_body(ea, rv, sv, wa, wr, ws, b1, w2, b2, out):
    acc = jnp.dot(ea[...], wa[...], preferred_element_type=jnp.float32)
    acc += jnp.dot(rv[...].astype(jnp.bfloat16), wr[...],
                   preferred_element_type=jnp.float32)
    acc += jnp.dot(sv[...].astype(jnp.bfloat16), ws[...],
                   preferred_element_type=jnp.float32)
    h = jnp.maximum(acc + b1[...], 0.0).astype(jnp.bfloat16)
    out[...] = jnp.dot(h, w2[...], preferred_element_type=jnp.float32) + b2[...]


def _edge_mlp(k, ea_bf, recv, send, wa, wr, ws, b1, w2, b2):
    n_blocks = C_E // EBLK
    base = k * n_blocks
    return pl.pallas_call(
        _edge_mlp_body,
        grid=(n_blocks,),
        in_specs=[
            pl.BlockSpec((EBLK, D_EDGE), lambda i, base=base: (base + i, 0)),
            pl.BlockSpec((EBLK, D_FEAT), lambda i: (i, 0)),
            pl.BlockSpec((EBLK, D_FEAT), lambda i: (i, 0)),
            pl.BlockSpec((D_EDGE, H), lambda i: (0, 0)),
            pl.BlockSpec((D_FEAT, H), lambda i: (0, 0)),
            pl.BlockSpec((D_FEAT, H), lambda i: (0, 0)),
            pl.BlockSpec((1, H), lambda i: (0, 0)),
            pl.BlockSpec((H, D_OUT), lambda i: (0, 0)),
            pl.BlockSpec((1, D_OUT), lambda i: (0, 0)),
        ],
        out_specs=pl.BlockSpec((EBLK, D_OUT), lambda i: (i, 0)),
        out_shape=jax.ShapeDtypeStruct((C_E, D_OUT), jnp.float32),
    )(ea_bf, recv, send, wa, wr, ws, b1, w2, b2)


def _node_mlp_body(a0, a1, a2, a3, a4, nd, wna, wnb, b1, w2, b2, out):
    agg = ((a0[0] + a0[1]) + (a1[0] + a1[1]) + (a2[0] + a2[1])
           + (a3[0] + a3[1]) + (a4[0] + a4[1])).astype(jnp.bfloat16)
    acc = jnp.dot(agg, wna[...], preferred_element_type=jnp.float32)
    acc += jnp.dot(nd[...].astype(jnp.bfloat16), wnb[...],
                   preferred_element_type=jnp.float32)
    h = jnp.maximum(acc + b1[...], 0.0).astype(jnp.bfloat16)
    out[...] = jnp.dot(h, w2[...], preferred_element_type=jnp.float32) + b2[...]


def _node_mlp(aggs, nodes, wna, wnb, b1, w2, b2):
    n_blocks = N_NODES // NBLK
    agg_spec = pl.BlockSpec((NC, NBLK, D_OUT), lambda i: (0, i, 0))
    return pl.pallas_call(
        _node_mlp_body,
        grid=(n_blocks,),
        in_specs=[agg_spec] * K_CH + [
            pl.BlockSpec((NBLK, D_FEAT), lambda i: (i, 0)),
            pl.BlockSpec((D_OUT, H), lambda i: (0, 0)),
            pl.BlockSpec((D_FEAT, H), lambda i: (0, 0)),
            pl.BlockSpec((1, H), lambda i: (0, 0)),
            pl.BlockSpec((H, D_OUT), lambda i: (0, 0)),
            pl.BlockSpec((1, D_OUT), lambda i: (0, 0)),
        ],
        out_specs=pl.BlockSpec((NBLK, D_OUT), lambda i: (i, 0)),
        out_shape=jax.ShapeDtypeStruct((N_NODES, D_OUT), jnp.float32),
    )(*aggs, nodes, wna, wnb, b1, w2, b2)


def kernel(nodes, edge_attr, senders, receivers,
           W_e1, b_e1, W_e2, b_e2, W_n1, b_n1, W_n2, b_n2):
    recv_i = receivers.astype(jnp.int32)
    send_i = senders.astype(jnp.int32)
    ea_bf = edge_attr.astype(jnp.bfloat16)

    wa = W_e1[:D_EDGE].astype(jnp.bfloat16)
    wr = W_e1[D_EDGE:D_EDGE + D_FEAT].astype(jnp.bfloat16)
    ws = W_e1[D_EDGE + D_FEAT:].astype(jnp.bfloat16)
    b1e = b_e1.reshape(1, H)
    w2e = W_e2.astype(jnp.bfloat16)
    b2e = b_e2.reshape(1, D_OUT)
    zeros = jnp.zeros((N_NODES, D_OUT), jnp.float32)

    ridx4 = recv_i.reshape(K_CH * NW, S_STEPS, SCH)
    eo_chunks = []
    agg_chunks = []
    for k in range(K_CH):
        recv_rows, send_rows = _GATHERS[k](nodes, recv_i, send_i)
        eo = _edge_mlp(k, ea_bf, recv_rows, send_rows,
                       wa, wr, ws, b1e, w2e, b2e)
        eo_chunks.append(eo)
        agg_chunks.append(_SCATTERS[k](eo, ridx4, zeros))

    edge_out = jnp.concatenate(eo_chunks, axis=0)

    wna = W_n1[:D_OUT].astype(jnp.bfloat16)
    wnb = W_n1[D_OUT:].astype(jnp.bfloat16)
    node_out = _node_mlp(
        agg_chunks, nodes, wna, wnb, b_n1.reshape(1, H),
        W_n2.astype(jnp.bfloat16), b_n2.reshape(1, D_OUT))

    return (node_out, edge_out)
